# baseline (speedup 1.0000x reference)
import jax
import jax.numpy as jnp
from jax.experimental import pallas as pl
from jax.experimental.pallas import tpu as pltpu

_BM = 400


def _linear_kernel(x_ref, w_ref, b_ref, h_ref):
    h_ref[...] = jax.lax.dot_general(
        x_ref[...], w_ref[...],
        (((1,), (1,)), ((), ())),
        preferred_element_type=jnp.float32,
    ) + b_ref[...]


def _agg_kernel(adj_ref, h_ref, out_ref):
    out_ref[...] = jnp.maximum(
        jnp.dot(adj_ref[...], h_ref[...], preferred_element_type=jnp.float32),
        0.0,
    )


def kernel(x, adj, W, b):
    N, _ = x.shape
    dout = W.shape[0]
    h = pl.pallas_call(
        _linear_kernel,
        out_shape=jax.ShapeDtypeStruct((N, dout), jnp.float32),
    )(x, W, b.reshape(1, dout))
    out = pl.pallas_call(
        _agg_kernel,
        grid=(N // _BM,),
        in_specs=[
            pl.BlockSpec((_BM, N), lambda i: (i, 0)),
            pl.BlockSpec((N, dout), lambda i: (0, 0)),
        ],
        out_specs=pl.BlockSpec((_BM, dout), lambda i: (i, 0)),
        out_shape=jax.ShapeDtypeStruct((N, dout), jnp.float32),
        compiler_params=pltpu.CompilerParams(
            dimension_semantics=("parallel",),
        ),
    )(adj, h)
    return out


# speedup vs baseline: 1.0228x; 1.0228x over previous
import jax
import jax.numpy as jnp
from jax.experimental import pallas as pl
from jax.experimental.pallas import tpu as pltpu

_BM = 400


def _linear_kernel(x_ref, w_ref, b_ref, h_ref):
    h = jax.lax.dot_general(
        x_ref[...], w_ref[...],
        (((1,), (1,)), ((), ())),
        preferred_element_type=jnp.float32,
    ) + b_ref[...]
    h_ref[...] = h.astype(jnp.bfloat16)


def _agg_kernel(adj_ref, h_ref, out_ref):
    out_ref[...] = jnp.maximum(
        jnp.dot(adj_ref[...].astype(jnp.bfloat16), h_ref[...],
                preferred_element_type=jnp.float32),
        0.0,
    )


def kernel(x, adj, W, b):
    N, _ = x.shape
    dout = W.shape[0]
    h = pl.pallas_call(
        _linear_kernel,
        out_shape=jax.ShapeDtypeStruct((N, dout), jnp.bfloat16),
    )(x, W, b.reshape(1, dout))
    out = pl.pallas_call(
        _agg_kernel,
        grid=(N // _BM,),
        in_specs=[
            pl.BlockSpec((_BM, N), lambda i: (i, 0)),
            pl.BlockSpec((N, dout), lambda i: (0, 0)),
        ],
        out_specs=pl.BlockSpec((_BM, dout), lambda i: (i, 0)),
        out_shape=jax.ShapeDtypeStruct((N, dout), jnp.float32),
        compiler_params=pltpu.CompilerParams(
            dimension_semantics=("parallel",),
        ),
    )(adj, h)
    return out


# speedup vs baseline: 1.0363x; 1.0132x over previous
import jax
import jax.numpy as jnp
from jax.experimental import pallas as pl
from jax.experimental.pallas import tpu as pltpu

_G = 2
_BM = 200


def _gcn_kernel(x_ref, w_ref, b_ref, *rest):
    adj_refs = rest[:_G]
    out_ref = rest[_G]
    h_ref = rest[_G + 1]

    @pl.when(pl.program_id(0) == 0)
    def _():
        h = jax.lax.dot_general(
            x_ref[...], w_ref[...],
            (((1,), (1,)), ((), ())),
            preferred_element_type=jnp.float32,
        ) + b_ref[...]
        h_ref[...] = h.astype(jnp.bfloat16)

    h = h_ref[...]
    for k in range(_G):
        out_ref[k * _BM:(k + 1) * _BM, :] = jnp.maximum(
            jnp.dot(adj_refs[k][...].astype(jnp.bfloat16), h,
                    preferred_element_type=jnp.float32),
            0.0,
        )


def kernel(x, adj, W, b):
    N, din = x.shape
    dout = W.shape[0]
    adj_specs = [
        pl.BlockSpec((_BM, N), lambda i, k=k: (i * _G + k, 0))
        for k in range(_G)
    ]
    return pl.pallas_call(
        _gcn_kernel,
        grid=(N // (_G * _BM),),
        in_specs=[
            pl.BlockSpec((N, din), lambda i: (0, 0)),
            pl.BlockSpec((dout, din), lambda i: (0, 0)),
            pl.BlockSpec((1, dout), lambda i: (0, 0)),
            *adj_specs,
        ],
        out_specs=pl.BlockSpec((_G * _BM, dout), lambda i: (i, 0)),
        out_shape=jax.ShapeDtypeStruct((N, dout), jnp.float32),
        scratch_shapes=[pltpu.VMEM((N, dout), jnp.bfloat16)],
        compiler_params=pltpu.CompilerParams(
            dimension_semantics=("arbitrary",),
        ),
    )(x, W, b.reshape(1, dout), *([adj] * _G))


# speedup vs baseline: 1.0477x; 1.0110x over previous
import jax
import jax.numpy as jnp
from jax.experimental import pallas as pl
from jax.experimental.pallas import tpu as pltpu

_BM = 200


def _gcn_kernel(x_ref, w_ref, b_ref, adj_ref, out_ref, h_ref):
    @pl.when(pl.program_id(0) == 0)
    def _():
        h = jax.lax.dot_general(
            x_ref[...], w_ref[...],
            (((1,), (1,)), ((), ())),
            preferred_element_type=jnp.float32,
        ) + b_ref[...]
        h_ref[...] = h.astype(jnp.bfloat16)

    out_ref[...] = jnp.maximum(
        jnp.dot(adj_ref[...].astype(jnp.bfloat16), h_ref[...],
                preferred_element_type=jnp.float32),
        0.0,
    )


def kernel(x, adj, W, b):
    N, din = x.shape
    dout = W.shape[0]
    return pl.pallas_call(
        _gcn_kernel,
        grid=(N // _BM,),
        in_specs=[
            pl.BlockSpec((N, din), lambda i: (0, 0)),
            pl.BlockSpec((dout, din), lambda i: (0, 0)),
            pl.BlockSpec((1, dout), lambda i: (0, 0)),
            pl.BlockSpec((_BM, N), lambda i: (i, 0)),
        ],
        out_specs=pl.BlockSpec((_BM, dout), lambda i: (i, 0)),
        out_shape=jax.ShapeDtypeStruct((N, dout), jnp.float32),
        scratch_shapes=[pltpu.VMEM((N, dout), jnp.bfloat16)],
        compiler_params=pltpu.CompilerParams(
            dimension_semantics=("arbitrary",),
        ),
    )(x, W, b.reshape(1, dout), adj)
